# baseline (speedup 1.0000x reference)
import functools

import jax
import jax.numpy as jnp
from jax.experimental import pallas as pl
from jax.experimental.pallas import tpu as pltpu

B = 1024
D_VIDEO = 2048
D_TEXT = 768
EMB = 64
POP = 100000
TOPK = 100

CHUNK = 2048
POP_PAD = 100352
NBLK = POP_PAD // CHUNK


def _fuse_body(xv_ref, xt_ref, wv_ref, wt_ref, bv_ref, bt_ref, out_ref):
    v = jax.lax.dot_general(xv_ref[...], wv_ref[...], (((1,), (0,)), ((), ())),
                            preferred_element_type=jnp.float32)
    t = jax.lax.dot_general(xt_ref[...], wt_ref[...], (((1,), (0,)), ((), ())),
                            preferred_element_type=jnp.float32)
    out_ref[...] = (v + bv_ref[...] + t + bt_ref[...]) * 0.5


def _sims_body(fused_ref, pop_ref, out_ref):
    j = pl.program_id(0)
    sims = jax.lax.dot_general(fused_ref[...], pop_ref[...], (((1,), (1,)), ((), ())),
                               preferred_element_type=jnp.float32)
    col = j * CHUNK + jax.lax.broadcasted_iota(jnp.int32, (B, CHUNK), 1)
    out_ref[...] = jnp.where(col < POP, sims, -jnp.inf)


def kernel(input_video, input_text, W_video, b_video, W_text, b_text, category_embs):
    fused = pl.pallas_call(
        _fuse_body,
        out_shape=jax.ShapeDtypeStruct((B, EMB), jnp.float32),
    )(input_video, input_text, W_video, W_text,
      b_video.reshape(1, EMB), b_text.reshape(1, EMB))
    fused = fused / (jnp.linalg.norm(fused, axis=-1, keepdims=True) + 1e-6)

    pop = category_embs / (jnp.linalg.norm(category_embs, axis=-1, keepdims=True) + 1e-6)
    pop = jnp.pad(pop, ((0, POP_PAD - POP), (0, 0)))
    sims = pl.pallas_call(
        _sims_body,
        grid=(NBLK,),
        in_specs=[
            pl.BlockSpec((B, EMB), lambda j: (0, 0)),
            pl.BlockSpec((CHUNK, EMB), lambda j: (j, 0)),
        ],
        out_specs=pl.BlockSpec((B, CHUNK), lambda j: (0, j)),
        out_shape=jax.ShapeDtypeStruct((B, POP_PAD), jnp.float32),
    )(fused, pop)

    _, knn_idx = jax.lax.top_k(sims, TOPK)
    return jnp.take(category_embs, knn_idx, axis=0)


# speedup vs baseline: 8.0463x; 8.0463x over previous
import functools

import jax
import jax.numpy as jnp
from jax import lax
from jax.experimental import pallas as pl
from jax.experimental.pallas import tpu as pltpu
from jax.experimental.pallas import tpu_sc as plsc

B = 1024
D_VIDEO = 2048
D_TEXT = 768
EMB = 64
POP = 100000
TOPK = 100

CHUNK = 2048
POP_PAD = 100352
NBLK = POP_PAD // CHUNK

NC = 2
NS = 16
NW = NC * NS
ROWS_PER_W = B // NW
NV = POP_PAD // 16
CAP = 4096
CAP2 = 512
NSEL = 112
NBIS = 22

NEG = -3e38
BIG = 2**30


def _fuse_body(xv_ref, xt_ref, wv_ref, wt_ref, bv_ref, bt_ref, out_ref):
    v = jax.lax.dot_general(xv_ref[...], wv_ref[...], (((1,), (0,)), ((), ())),
                            preferred_element_type=jnp.float32)
    t = jax.lax.dot_general(xt_ref[...], wt_ref[...], (((1,), (0,)), ((), ())),
                            preferred_element_type=jnp.float32)
    out_ref[...] = (v + bv_ref[...] + t + bt_ref[...]) * 0.5


def _sims_body(fused_ref, pop_ref, sims_ref, tau_ref, m_scr):
    j = pl.program_id(0)
    sims = jax.lax.dot_general(fused_ref[...], pop_ref[...], (((1,), (1,)), ((), ())),
                               preferred_element_type=jnp.float32)
    col = j * CHUNK + jax.lax.broadcasted_iota(jnp.int32, (B, CHUNK), 1)
    sims = jnp.where(col < POP, sims, NEG)
    sims_ref[...] = sims

    cm = sims[:, 0:128]
    for s in range(1, CHUNK // 128):
        cm = jnp.maximum(cm, sims[:, s * 128:(s + 1) * 128])

    @pl.when(j == 0)
    def _():
        m_scr[...] = cm

    @pl.when(j > 0)
    def _():
        m_scr[...] = jnp.maximum(m_scr[...], cm)

    @pl.when(j == NBLK - 1)
    def _():
        tau = jnp.min(m_scr[...], axis=1, keepdims=True)
        tau_ref[...] = jnp.broadcast_to(tau, (B, 128))


def _make_sc_retrieve():
    mesh = plsc.VectorSubcoreMesh(core_axis_name="c", subcore_axis_name="s")

    @functools.partial(
        pl.kernel, mesh=mesh,
        out_type=jax.ShapeDtypeStruct((B, TOPK, EMB), jnp.float32),
        compiler_params=pltpu.CompilerParams(needs_layout_passes=False,
                                             use_tc_tiling_on_sc=False),
        scratch_types=[
            pltpu.VMEM((POP_PAD,), jnp.float32),
            pltpu.VMEM((ROWS_PER_W, 16), jnp.float32),
            pltpu.VMEM((CAP,), jnp.float32),
            pltpu.VMEM((CAP,), jnp.int32),
            pltpu.VMEM((CAP2,), jnp.float32),
            pltpu.VMEM((CAP2,), jnp.int32),
            pltpu.VMEM((NSEL,), jnp.int32),
            pltpu.VMEM((NSEL, EMB), jnp.float32),
            pltpu.SemaphoreType.DMA,
        ],
    )
    def sc_retrieve(sims_hbm, tau_hbm, cat_hbm, out_hbm,
                    sims_v, tau_v, cv, ci, cv2, ci2, knn_v, emb_v, sem):
        wid = lax.axis_index("s") * NC + lax.axis_index("c")
        base_row = wid * ROWS_PER_W
        pltpu.sync_copy(tau_hbm.at[pl.ds(base_row, ROWS_PER_W)], tau_v)
        iota16 = lax.iota(jnp.int32, 16)
        ones16 = iota16 >= 0

        def do_row(r, _):
            b = base_row + r
            pltpu.sync_copy(sims_hbm.at[b], sims_v)
            tsp = tau_v[r]
            tau_s = tsp[0]

            def filt(i, carry):
                off, rmax = carry
                v = sims_v[pl.ds(i * 16, 16)]
                mask = v >= tsp
                off_use = jnp.minimum(off, CAP - 16)
                plsc.store_compressed(cv.at[pl.ds(off_use, 16)], v, mask=mask)
                plsc.store_compressed(ci.at[pl.ds(off_use, 16)],
                                      i * 16 + iota16, mask=mask)
                cnt = plsc.all_reduce_population_count(mask)
                return off + cnt[0], jnp.maximum(rmax, v)

            c, rmax_v = lax.fori_loop(
                0, NV, filt, (jnp.int32(0), jnp.full((16,), NEG, jnp.float32)))
            c = jnp.minimum(c, CAP - 16)
            plsc.store_compressed(cv.at[pl.ds(c, 16)],
                                  jnp.full((16,), NEG, jnp.float32), mask=ones16)
            nvec = (c + 15) // 16
            rmax = jnp.max(rmax_v)

            def bstep(_, lohi):
                lo, hi = lohi
                mid = 0.5 * (lo + hi)
                mids = jnp.full((16,), mid, jnp.float32)

                def cbody(i, acc):
                    v = cv[pl.ds(i * 16, 16)]
                    return acc + plsc.all_reduce_population_count(v >= mids)

                cnt = lax.fori_loop(0, nvec, cbody, jnp.zeros((16,), jnp.int32))[0]
                ok = cnt >= NSEL
                return jnp.where(ok, mid, lo), jnp.where(ok, hi, mid)

            lo, _ = lax.fori_loop(0, NBIS, bstep, (tau_s, rmax + jnp.float32(1.0)))

            los = jnp.full((16,), lo, jnp.float32)

            def filt2(i, off):
                v = cv[pl.ds(i * 16, 16)]
                ix = ci[pl.ds(i * 16, 16)]
                mask = v >= los
                off_use = jnp.minimum(off, CAP2 - 16)
                plsc.store_compressed(cv2.at[pl.ds(off_use, 16)], v, mask=mask)
                plsc.store_compressed(ci2.at[pl.ds(off_use, 16)], ix, mask=mask)
                return off + plsc.all_reduce_population_count(mask)[0]

            c2 = lax.fori_loop(0, nvec, filt2, jnp.int32(0))
            c2 = jnp.minimum(c2, CAP2 - 16)
            plsc.store_compressed(cv2.at[pl.ds(c2, 16)],
                                  jnp.full((16,), NEG, jnp.float32), mask=ones16)
            nvec2 = (c2 + 15) // 16

            pv = rmax + jnp.float32(1.0)
            pidx = jnp.int32(-1)
            for g in range(NSEL // 16):
                def ext(k, carry):
                    pv, pidx, selv = carry
                    pvs = jnp.full((16,), pv, jnp.float32)
                    pis = jnp.full((16,), pidx, jnp.int32)

                    def scan(i, mi):
                        mx, ix = mi
                        v = cv2[pl.ds(i * 16, 16)]
                        d = ci2[pl.ds(i * 16, 16)]
                        elig = (v < pvs) | ((v == pvs) & (d > pis))
                        better = elig & ((v > mx) | ((v == mx) & (d < ix)))
                        return jnp.where(better, v, mx), jnp.where(better, d, ix)

                    mx, ix = lax.fori_loop(
                        0, nvec2, scan,
                        (jnp.full((16,), NEG, jnp.float32),
                         jnp.full((16,), BIG, jnp.int32)))
                    m = jnp.max(mx)
                    ixm = jnp.where(mx == jnp.full((16,), m, jnp.float32), ix,
                                    jnp.full((16,), BIG, jnp.int32))
                    sel = jnp.min(ixm)
                    selv = jnp.where(iota16 == jnp.full((16,), k, jnp.int32),
                                     jnp.full((16,), sel, jnp.int32), selv)
                    return m, sel, selv

                pv, pidx, selv = lax.fori_loop(
                    0, 16, ext, (pv, pidx, jnp.zeros((16,), jnp.int32)))
                knn_v[pl.ds(g * 16, 16)] = selv

            pltpu.async_copy(cat_hbm.at[knn_v], emb_v, sem).wait()
            pltpu.sync_copy(emb_v.at[pl.ds(0, TOPK)], out_hbm.at[b])
            return 0

        lax.fori_loop(0, ROWS_PER_W, do_row, 0)

    return sc_retrieve


_sc_retrieve = _make_sc_retrieve()


def kernel(input_video, input_text, W_video, b_video, W_text, b_text, category_embs):
    fused = pl.pallas_call(
        _fuse_body,
        out_shape=jax.ShapeDtypeStruct((B, EMB), jnp.float32),
    )(input_video, input_text, W_video, W_text,
      b_video.reshape(1, EMB), b_text.reshape(1, EMB))
    fused = fused / (jnp.linalg.norm(fused, axis=-1, keepdims=True) + 1e-6)

    pop = category_embs / (jnp.linalg.norm(category_embs, axis=-1, keepdims=True) + 1e-6)
    pop = jnp.pad(pop, ((0, POP_PAD - POP), (0, 0)))
    sims, tau2d = pl.pallas_call(
        _sims_body,
        grid=(NBLK,),
        in_specs=[
            pl.BlockSpec((B, EMB), lambda j: (0, 0)),
            pl.BlockSpec((CHUNK, EMB), lambda j: (j, 0)),
        ],
        out_specs=[
            pl.BlockSpec((B, CHUNK), lambda j: (0, j)),
            pl.BlockSpec((B, 128), lambda j: (0, 0)),
        ],
        out_shape=[
            jax.ShapeDtypeStruct((B, POP_PAD), jnp.float32),
            jax.ShapeDtypeStruct((B, 128), jnp.float32),
        ],
        scratch_shapes=[pltpu.VMEM((B, 128), jnp.float32)],
    )(fused, pop)

    tau = tau2d[:, :16]
    return _sc_retrieve(sims, tau, category_embs)


# speedup vs baseline: 8.3922x; 1.0430x over previous
import functools

import jax
import jax.numpy as jnp
from jax import lax
from jax.experimental import pallas as pl
from jax.experimental.pallas import tpu as pltpu
from jax.experimental.pallas import tpu_sc as plsc

B = 1024
D_VIDEO = 2048
D_TEXT = 768
EMB = 64
POP = 100000
TOPK = 100

CHUNK = 2048
POP_PAD = 100352
NBLK = POP_PAD // CHUNK

NC = 2
NS = 16
NW = NC * NS
ROWS_PER_W = B // NW
NSC = 8
SCH = POP_PAD // NSC
NVC = SCH // 16
CAP = 4096
CAP2 = 512
NSEL = 112
NBIS = 18

NEG = -3e38
BIG = 2**30


def _fuse_body(xv_ref, xt_ref, wv_ref, wt_ref, bv_ref, bt_ref, out_ref):
    v = jax.lax.dot_general(xv_ref[...], wv_ref[...], (((1,), (0,)), ((), ())),
                            preferred_element_type=jnp.float32)
    t = jax.lax.dot_general(xt_ref[...], wt_ref[...], (((1,), (0,)), ((), ())),
                            preferred_element_type=jnp.float32)
    out_ref[...] = (v + bv_ref[...] + t + bt_ref[...]) * 0.5


def _sims_body(fused_ref, pop_ref, sims_ref, tau_ref, m_scr):
    j = pl.program_id(0)
    sims = jax.lax.dot_general(fused_ref[...], pop_ref[...], (((1,), (1,)), ((), ())),
                               preferred_element_type=jnp.float32)
    col = j * CHUNK + jax.lax.broadcasted_iota(jnp.int32, (B, CHUNK), 1)
    sims = jnp.where(col < POP, sims, NEG)
    sims_ref[...] = sims

    cm = sims[:, 0:128]
    for s in range(1, CHUNK // 128):
        cm = jnp.maximum(cm, sims[:, s * 128:(s + 1) * 128])

    @pl.when(j == 0)
    def _():
        m_scr[...] = cm

    @pl.when(j > 0)
    def _():
        m_scr[...] = jnp.maximum(m_scr[...], cm)

    @pl.when(j == NBLK - 1)
    def _():
        tau = jnp.min(m_scr[...], axis=1, keepdims=True)
        tau_ref[...] = jnp.broadcast_to(tau, (B, 128))


def _make_sc_retrieve():
    mesh = plsc.VectorSubcoreMesh(core_axis_name="c", subcore_axis_name="s")

    @functools.partial(
        pl.kernel, mesh=mesh,
        out_type=jax.ShapeDtypeStruct((B, TOPK, EMB), jnp.float32),
        compiler_params=pltpu.CompilerParams(needs_layout_passes=False,
                                             use_tc_tiling_on_sc=False),
        scratch_types=[
            pltpu.VMEM((2, SCH), jnp.float32),
            pltpu.VMEM((ROWS_PER_W, 16), jnp.float32),
            pltpu.VMEM((CAP,), jnp.float32),
            pltpu.VMEM((CAP,), jnp.int32),
            pltpu.VMEM((CAP2,), jnp.float32),
            pltpu.VMEM((CAP2,), jnp.int32),
            pltpu.VMEM((NSEL,), jnp.int32),
            pltpu.VMEM((NSEL, EMB), jnp.float32),
            pltpu.SemaphoreType.DMA,
            pltpu.SemaphoreType.DMA,
        ],
    )
    def sc_retrieve(sims_hbm, tau_hbm, cat_hbm, out_hbm,
                    ring, tau_v, cv, ci, cv2, ci2, knn_v, emb_v, sem0, sem1):
        wid = lax.axis_index("s") * NC + lax.axis_index("c")
        base_row = wid * ROWS_PER_W
        pltpu.sync_copy(tau_hbm.at[pl.ds(base_row, ROWS_PER_W)], tau_v)
        iota16 = lax.iota(jnp.int32, 16)
        ones16 = iota16 >= 0
        sems = (sem0, sem1)

        def do_row(r, _):
            b = base_row + r
            tsp = tau_v[r]
            tau_s = tsp[0]

            dmas = [pltpu.async_copy(sims_hbm.at[b, pl.ds(0, SCH)],
                                     ring.at[0], sems[0])]
            off = jnp.int32(0)
            rmax_v = jnp.full((16,), NEG, jnp.float32)
            for g in range(NSC):
                dmas[g].wait()
                if g + 1 < NSC:
                    dmas.append(pltpu.async_copy(
                        sims_hbm.at[b, pl.ds((g + 1) * SCH, SCH)],
                        ring.at[(g + 1) % 2], sems[(g + 1) % 2]))
                buf = ring.at[g % 2]
                gbase = g * SCH

                def filt(i, carry):
                    off, rmax = carry
                    v = buf[pl.ds(i * 16, 16)]
                    mask = v >= tsp
                    off_use = jnp.minimum(off, CAP - 16)
                    plsc.store_compressed(cv.at[pl.ds(off_use, 16)], v, mask=mask)
                    plsc.store_compressed(ci.at[pl.ds(off_use, 16)],
                                          gbase + i * 16 + iota16, mask=mask)
                    cnt = plsc.all_reduce_population_count(mask)
                    return off + cnt[0], jnp.maximum(rmax, v)

                off, rmax_v = lax.fori_loop(0, NVC, filt, (off, rmax_v))

            c = jnp.minimum(off, CAP - 16)
            plsc.store_compressed(cv.at[pl.ds(c, 16)],
                                  jnp.full((16,), NEG, jnp.float32), mask=ones16)
            nvec = (c + 15) // 16
            rmax = jnp.max(rmax_v)

            def bstep(_, lohi):
                lo, hi = lohi
                mid = 0.5 * (lo + hi)
                mids = jnp.full((16,), mid, jnp.float32)

                def cbody(i, acc):
                    v = cv[pl.ds(i * 16, 16)]
                    return acc + plsc.all_reduce_population_count(v >= mids)

                cnt = lax.fori_loop(0, nvec, cbody, jnp.zeros((16,), jnp.int32))[0]
                ok = cnt >= NSEL
                return jnp.where(ok, mid, lo), jnp.where(ok, hi, mid)

            lo, _ = lax.fori_loop(0, NBIS, bstep, (tau_s, rmax + jnp.float32(1.0)))

            los = jnp.full((16,), lo, jnp.float32)

            def filt2(i, off2):
                v = cv[pl.ds(i * 16, 16)]
                ix = ci[pl.ds(i * 16, 16)]
                mask = v >= los
                off_use = jnp.minimum(off2, CAP2 - 16)
                plsc.store_compressed(cv2.at[pl.ds(off_use, 16)], v, mask=mask)
                plsc.store_compressed(ci2.at[pl.ds(off_use, 16)], ix, mask=mask)
                return off2 + plsc.all_reduce_population_count(mask)[0]

            c2 = lax.fori_loop(0, nvec, filt2, jnp.int32(0))
            c2 = jnp.minimum(c2, CAP2 - 16)
            plsc.store_compressed(cv2.at[pl.ds(c2, 16)],
                                  jnp.full((16,), NEG, jnp.float32), mask=ones16)
            nvec2 = (c2 + 15) // 16

            pv = rmax + jnp.float32(1.0)
            pidx = jnp.int32(-1)
            for g in range(NSEL // 16):
                def ext(k, carry):
                    pv, pidx, selv = carry
                    pvs = jnp.full((16,), pv, jnp.float32)
                    pis = jnp.full((16,), pidx, jnp.int32)

                    def scan(i, mi):
                        mx, ix = mi
                        v = cv2[pl.ds(i * 16, 16)]
                        d = ci2[pl.ds(i * 16, 16)]
                        elig = (v < pvs) | ((v == pvs) & (d > pis))
                        better = elig & ((v > mx) | ((v == mx) & (d < ix)))
                        return jnp.where(better, v, mx), jnp.where(better, d, ix)

                    mx, ix = lax.fori_loop(
                        0, nvec2, scan,
                        (jnp.full((16,), NEG, jnp.float32),
                         jnp.full((16,), BIG, jnp.int32)))
                    m = jnp.max(mx)
                    ixm = jnp.where(mx == jnp.full((16,), m, jnp.float32), ix,
                                    jnp.full((16,), BIG, jnp.int32))
                    sel = jnp.min(ixm)
                    selv = jnp.where(iota16 == jnp.full((16,), k, jnp.int32),
                                     jnp.full((16,), sel, jnp.int32), selv)
                    return m, sel, selv

                pv, pidx, selv = lax.fori_loop(
                    0, 16, ext, (pv, pidx, jnp.zeros((16,), jnp.int32)))
                knn_v[pl.ds(g * 16, 16)] = selv

            pltpu.async_copy(cat_hbm.at[knn_v], emb_v, sem0).wait()
            pltpu.sync_copy(emb_v.at[pl.ds(0, TOPK)], out_hbm.at[b])
            return 0

        lax.fori_loop(0, ROWS_PER_W, do_row, 0)

    return sc_retrieve


_sc_retrieve = _make_sc_retrieve()


def kernel(input_video, input_text, W_video, b_video, W_text, b_text, category_embs):
    fused = pl.pallas_call(
        _fuse_body,
        out_shape=jax.ShapeDtypeStruct((B, EMB), jnp.float32),
    )(input_video, input_text, W_video, W_text,
      b_video.reshape(1, EMB), b_text.reshape(1, EMB))
    fused = fused / (jnp.linalg.norm(fused, axis=-1, keepdims=True) + 1e-6)

    pop = category_embs / (jnp.linalg.norm(category_embs, axis=-1, keepdims=True) + 1e-6)
    pop = jnp.pad(pop, ((0, POP_PAD - POP), (0, 0)))
    sims, tau2d = pl.pallas_call(
        _sims_body,
        grid=(NBLK,),
        in_specs=[
            pl.BlockSpec((B, EMB), lambda j: (0, 0)),
            pl.BlockSpec((CHUNK, EMB), lambda j: (j, 0)),
        ],
        out_specs=[
            pl.BlockSpec((B, CHUNK), lambda j: (0, j)),
            pl.BlockSpec((B, 128), lambda j: (0, 0)),
        ],
        out_shape=[
            jax.ShapeDtypeStruct((B, POP_PAD), jnp.float32),
            jax.ShapeDtypeStruct((B, 128), jnp.float32),
        ],
        scratch_shapes=[pltpu.VMEM((B, 128), jnp.float32)],
    )(fused, pop)

    tau = tau2d[:, :16]
    return _sc_retrieve(sims, tau, category_embs)


# speedup vs baseline: 12.1049x; 1.4424x over previous
import functools

import jax
import jax.numpy as jnp
from jax import lax
from jax.experimental import pallas as pl
from jax.experimental.pallas import tpu as pltpu
from jax.experimental.pallas import tpu_sc as plsc

B = 1024
D_VIDEO = 2048
D_TEXT = 768
EMB = 64
POP = 100000
TOPK = 100

CHUNK = 2048
POP_PAD = 100352
NBLK = POP_PAD // CHUNK

NC = 2
NS = 16
NW = NC * NS
ROWS_PER_W = B // NW
NSC = 8
SCH = POP_PAD // NSC
NVC = SCH // 16
CAP = 4096
CAP2 = 512
NSEL = 112
NBIS = 18

NEG = -3e38
BIG = 2**30


def _fuse_body(xv_ref, xt_ref, wv_ref, wt_ref, bv_ref, bt_ref, out_ref):
    v = jax.lax.dot_general(xv_ref[...], wv_ref[...], (((1,), (0,)), ((), ())),
                            preferred_element_type=jnp.float32)
    t = jax.lax.dot_general(xt_ref[...], wt_ref[...], (((1,), (0,)), ((), ())),
                            preferred_element_type=jnp.float32)
    out_ref[...] = (v + bv_ref[...] + t + bt_ref[...]) * 0.5


def _sims_body(fused_ref, pop_ref, sims_ref, tau_ref, m_scr):
    j = pl.program_id(0)
    sims = jax.lax.dot_general(fused_ref[...], pop_ref[...], (((1,), (1,)), ((), ())),
                               preferred_element_type=jnp.float32)
    col = j * CHUNK + jax.lax.broadcasted_iota(jnp.int32, (B, CHUNK), 1)
    sims = jnp.where(col < POP, sims, NEG)
    sims_ref[...] = sims

    cm = sims[:, 0:128]
    for s in range(1, CHUNK // 128):
        cm = jnp.maximum(cm, sims[:, s * 128:(s + 1) * 128])

    @pl.when(j == 0)
    def _():
        m_scr[...] = cm

    @pl.when(j > 0)
    def _():
        m_scr[...] = jnp.maximum(m_scr[...], cm)

    @pl.when(j == NBLK - 1)
    def _():
        tau = jnp.min(m_scr[...], axis=1, keepdims=True)
        tau_ref[...] = jnp.broadcast_to(tau, (B, 128))


def _make_sc_retrieve():
    mesh = plsc.VectorSubcoreMesh(core_axis_name="c", subcore_axis_name="s")

    @functools.partial(
        pl.kernel, mesh=mesh,
        out_type=jax.ShapeDtypeStruct((B, TOPK, EMB), jnp.float32),
        compiler_params=pltpu.CompilerParams(needs_layout_passes=False,
                                             use_tc_tiling_on_sc=False),
        scratch_types=[
            pltpu.VMEM((2, SCH), jnp.float32),
            pltpu.VMEM((ROWS_PER_W, 16), jnp.float32),
            pltpu.VMEM((CAP,), jnp.float32),
            pltpu.VMEM((CAP,), jnp.int32),
            pltpu.VMEM((CAP2,), jnp.float32),
            pltpu.VMEM((CAP2,), jnp.int32),
            pltpu.VMEM((NSEL,), jnp.int32),
            pltpu.VMEM((NSEL, EMB), jnp.float32),
            pltpu.SemaphoreType.DMA,
            pltpu.SemaphoreType.DMA,
        ],
    )
    def sc_retrieve(sims_hbm, tau_hbm, cat_hbm, out_hbm,
                    ring, tau_v, cv, ci, cv2, ci2, knn_v, emb_v, sem0, sem1):
        wid = lax.axis_index("s") * NC + lax.axis_index("c")
        base_row = wid * ROWS_PER_W
        pltpu.sync_copy(tau_hbm.at[pl.ds(base_row, ROWS_PER_W)], tau_v)
        iota16 = lax.iota(jnp.int32, 16)
        ones16 = iota16 >= 0
        sems = (sem0, sem1)

        def do_row(r, _):
            b = base_row + r
            tsp = tau_v[r]
            tau_s = tsp[0]

            dmas = [pltpu.async_copy(sims_hbm.at[b, pl.ds(0, SCH)],
                                     ring.at[0], sems[0])]
            off = jnp.int32(0)
            rmax_v = jnp.full((16,), NEG, jnp.float32)
            for g in range(NSC):
                dmas[g].wait()
                if g + 1 < NSC:
                    dmas.append(pltpu.async_copy(
                        sims_hbm.at[b, pl.ds((g + 1) * SCH, SCH)],
                        ring.at[(g + 1) % 2], sems[(g + 1) % 2]))
                buf = ring.at[g % 2]
                gbase = g * SCH

                def filt(i, carry):
                    off0, rmax = carry
                    vs, masks, cnts = [], [], []
                    for u in range(4):
                        v = buf[pl.ds((i * 4 + u) * 16, 16)]
                        mask = v >= tsp
                        rmax = jnp.maximum(rmax, v)
                        cnts.append(plsc.all_reduce_population_count(mask))
                        vs.append(v)
                        masks.append(mask)
                    offs = [off0]
                    for u in range(3):
                        offs.append(offs[u] + cnts[u][0])
                    for u in range(4):
                        off_use = jnp.minimum(offs[u], CAP - 16)
                        plsc.store_compressed(cv.at[pl.ds(off_use, 16)],
                                              vs[u], mask=masks[u])
                        plsc.store_compressed(ci.at[pl.ds(off_use, 16)],
                                              gbase + (i * 4 + u) * 16 + iota16,
                                              mask=masks[u])
                    return offs[3] + cnts[3][0], rmax

                off, rmax_v = lax.fori_loop(0, NVC // 4, filt, (off, rmax_v))

            c = jnp.minimum(off, CAP - 16)
            plsc.store_compressed(cv.at[pl.ds(c, 16)],
                                  jnp.full((16,), NEG, jnp.float32), mask=ones16)
            nvec = (c + 15) // 16
            rmax = jnp.max(rmax_v)

            def bstep(_, lohi):
                lo, hi = lohi
                mid = 0.5 * (lo + hi)
                mids = jnp.full((16,), mid, jnp.float32)

                def cbody(i, acc):
                    v = cv[pl.ds(i * 16, 16)]
                    return acc + plsc.all_reduce_population_count(v >= mids)

                cnt = lax.fori_loop(0, nvec, cbody, jnp.zeros((16,), jnp.int32))[0]
                ok = cnt >= NSEL
                return jnp.where(ok, mid, lo), jnp.where(ok, hi, mid)

            lo, _ = lax.fori_loop(0, NBIS, bstep, (tau_s, rmax + jnp.float32(1.0)))

            los = jnp.full((16,), lo, jnp.float32)

            def filt2(i, off2):
                v = cv[pl.ds(i * 16, 16)]
                ix = ci[pl.ds(i * 16, 16)]
                mask = v >= los
                off_use = jnp.minimum(off2, CAP2 - 16)
                plsc.store_compressed(cv2.at[pl.ds(off_use, 16)], v, mask=mask)
                plsc.store_compressed(ci2.at[pl.ds(off_use, 16)], ix, mask=mask)
                return off2 + plsc.all_reduce_population_count(mask)[0]

            c2 = lax.fori_loop(0, nvec, filt2, jnp.int32(0))
            c2 = jnp.minimum(c2, CAP2 - 16)
            plsc.store_compressed(cv2.at[pl.ds(c2, 16)],
                                  jnp.full((16,), NEG, jnp.float32), mask=ones16)
            nvec2 = (c2 + 15) // 16

            pv = rmax + jnp.float32(1.0)
            pidx = jnp.int32(-1)
            for g in range(NSEL // 16):
                def ext(k, carry):
                    pv, pidx, selv = carry
                    pvs = jnp.full((16,), pv, jnp.float32)
                    pis = jnp.full((16,), pidx, jnp.int32)

                    def scan(i, mi):
                        mx, ix = mi
                        v = cv2[pl.ds(i * 16, 16)]
                        d = ci2[pl.ds(i * 16, 16)]
                        elig = (v < pvs) | ((v == pvs) & (d > pis))
                        better = elig & ((v > mx) | ((v == mx) & (d < ix)))
                        return jnp.where(better, v, mx), jnp.where(better, d, ix)

                    mx, ix = lax.fori_loop(
                        0, nvec2, scan,
                        (jnp.full((16,), NEG, jnp.float32),
                         jnp.full((16,), BIG, jnp.int32)))
                    m = jnp.max(mx)
                    ixm = jnp.where(mx == jnp.full((16,), m, jnp.float32), ix,
                                    jnp.full((16,), BIG, jnp.int32))
                    sel = jnp.min(ixm)
                    selv = jnp.where(iota16 == jnp.full((16,), k, jnp.int32),
                                     jnp.full((16,), sel, jnp.int32), selv)
                    return m, sel, selv

                pv, pidx, selv = lax.fori_loop(
                    0, 16, ext, (pv, pidx, jnp.zeros((16,), jnp.int32)))
                knn_v[pl.ds(g * 16, 16)] = selv

            pltpu.async_copy(cat_hbm.at[knn_v], emb_v, sem0).wait()
            pltpu.sync_copy(emb_v.at[pl.ds(0, TOPK)], out_hbm.at[b])
            return 0

        lax.fori_loop(0, ROWS_PER_W, do_row, 0)

    return sc_retrieve


_sc_retrieve = _make_sc_retrieve()


def kernel(input_video, input_text, W_video, b_video, W_text, b_text, category_embs):
    fused = pl.pallas_call(
        _fuse_body,
        out_shape=jax.ShapeDtypeStruct((B, EMB), jnp.float32),
    )(input_video, input_text, W_video, W_text,
      b_video.reshape(1, EMB), b_text.reshape(1, EMB))
    fused = fused / (jnp.linalg.norm(fused, axis=-1, keepdims=True) + 1e-6)

    pop = category_embs / (jnp.linalg.norm(category_embs, axis=-1, keepdims=True) + 1e-6)
    pop = jnp.pad(pop, ((0, POP_PAD - POP), (0, 0)))
    sims, tau2d = pl.pallas_call(
        _sims_body,
        grid=(NBLK,),
        in_specs=[
            pl.BlockSpec((B, EMB), lambda j: (0, 0)),
            pl.BlockSpec((CHUNK, EMB), lambda j: (j, 0)),
        ],
        out_specs=[
            pl.BlockSpec((B, CHUNK), lambda j: (0, j)),
            pl.BlockSpec((B, 128), lambda j: (0, 0)),
        ],
        out_shape=[
            jax.ShapeDtypeStruct((B, POP_PAD), jnp.float32),
            jax.ShapeDtypeStruct((B, 128), jnp.float32),
        ],
        scratch_shapes=[pltpu.VMEM((B, 128), jnp.float32)],
    )(fused, pop)

    tau = tau2d[:, :16]
    return _sc_retrieve(sims, tau, category_embs)


# speedup vs baseline: 12.8819x; 1.0642x over previous
import functools

import jax
import jax.numpy as jnp
from jax import lax
from jax.experimental import pallas as pl
from jax.experimental.pallas import tpu as pltpu
from jax.experimental.pallas import tpu_sc as plsc

B = 1024
D_VIDEO = 2048
D_TEXT = 768
EMB = 64
POP = 100000
TOPK = 100

CHUNK = 2048
POP_PAD = 100352
NBLK = POP_PAD // CHUNK

NC = 2
NS = 16
NW = NC * NS
ROWS_PER_W = B // NW
NSC = 8
SCH = POP_PAD // NSC
NVC = SCH // 16
CAP = 4096
CAP2 = 512
NSEL = 112
NBIS = 18

NEG = -3e38
BIG = 2**30


def _fuse_body(xv_ref, xt_ref, wv_ref, wt_ref, bv_ref, bt_ref, out_ref):
    v = jax.lax.dot_general(xv_ref[...], wv_ref[...], (((1,), (0,)), ((), ())),
                            preferred_element_type=jnp.float32)
    t = jax.lax.dot_general(xt_ref[...], wt_ref[...], (((1,), (0,)), ((), ())),
                            preferred_element_type=jnp.float32)
    out_ref[...] = (v + bv_ref[...] + t + bt_ref[...]) * 0.5


def _sims_body(fused_ref, pop_ref, sims_ref, tau_ref, m_scr):
    j = pl.program_id(0)
    sims = jax.lax.dot_general(fused_ref[...], pop_ref[...], (((1,), (1,)), ((), ())),
                               preferred_element_type=jnp.float32)
    col = j * CHUNK + jax.lax.broadcasted_iota(jnp.int32, (B, CHUNK), 1)
    sims = jnp.where(col < POP, sims, NEG)
    sims_ref[...] = sims

    cm = sims[:, 0:128]
    for s in range(1, CHUNK // 128):
        cm = jnp.maximum(cm, sims[:, s * 128:(s + 1) * 128])

    @pl.when(j == 0)
    def _():
        m_scr[...] = cm

    @pl.when(j > 0)
    def _():
        m_scr[...] = jnp.maximum(m_scr[...], cm)

    @pl.when(j == NBLK - 1)
    def _():
        tau = jnp.min(m_scr[...], axis=1, keepdims=True)
        tau_ref[...] = jnp.broadcast_to(tau, (B, 128))


def _make_sc_retrieve():
    mesh = plsc.VectorSubcoreMesh(core_axis_name="c", subcore_axis_name="s")

    @functools.partial(
        pl.kernel, mesh=mesh,
        out_type=jax.ShapeDtypeStruct((B, TOPK, EMB), jnp.float32),
        compiler_params=pltpu.CompilerParams(needs_layout_passes=False,
                                             use_tc_tiling_on_sc=False),
        scratch_types=[
            pltpu.VMEM((4, SCH), jnp.float32),
            pltpu.VMEM((ROWS_PER_W, 16), jnp.float32),
            pltpu.VMEM((CAP,), jnp.float32),
            pltpu.VMEM((CAP,), jnp.int32),
            pltpu.VMEM((CAP2,), jnp.float32),
            pltpu.VMEM((CAP2,), jnp.int32),
            pltpu.VMEM((NSEL,), jnp.int32),
            pltpu.VMEM((NSEL, EMB), jnp.float32),
            pltpu.SemaphoreType.DMA,
            pltpu.SemaphoreType.DMA,
            pltpu.SemaphoreType.DMA,
            pltpu.SemaphoreType.DMA,
            pltpu.SemaphoreType.DMA,
        ],
    )
    def sc_retrieve(sims_hbm, tau_hbm, cat_hbm, out_hbm,
                    ring, tau_v, cv, ci, cv2, ci2, knn_v, emb_v,
                    sem0, sem1, sem2, sem3, gsem):
        wid = lax.axis_index("s") * NC + lax.axis_index("c")
        base_row = wid * ROWS_PER_W
        pltpu.sync_copy(tau_hbm.at[pl.ds(base_row, ROWS_PER_W)], tau_v)
        iota16 = lax.iota(jnp.int32, 16)
        ones16 = iota16 >= 0
        sems = (sem0, sem1, sem2, sem3)

        pltpu.async_copy(sims_hbm.at[base_row, pl.ds(0, SCH)], ring.at[0], sem0)
        pltpu.async_copy(sims_hbm.at[base_row, pl.ds(SCH, SCH)], ring.at[1], sem1)

        def do_row(r, _):
            b = base_row + r
            tsp = tau_v[r]
            tau_s = tsp[0]

            off = jnp.int32(0)
            rmax_v = jnp.full((16,), NEG, jnp.float32)
            for g in range(NSC):
                pltpu.make_async_copy(sims_hbm.at[b, pl.ds(g * SCH, SCH)],
                                      ring.at[g % 4], sems[g % 4]).wait()
                if g + 2 < NSC:
                    pltpu.async_copy(sims_hbm.at[b, pl.ds((g + 2) * SCH, SCH)],
                                     ring.at[(g + 2) % 4], sems[(g + 2) % 4])
                else:
                    g2 = g + 2 - NSC

                    @pl.when(r + 1 < ROWS_PER_W)
                    def _():
                        pltpu.async_copy(
                            sims_hbm.at[b + 1, pl.ds(g2 * SCH, SCH)],
                            ring.at[(g + 2) % 4], sems[(g + 2) % 4])
                buf = ring.at[g % 4]
                gbase = g * SCH

                def filt(i, carry):
                    off0, rmax = carry
                    vs, masks, cnts = [], [], []
                    for u in range(4):
                        v = buf[pl.ds((i * 4 + u) * 16, 16)]
                        mask = v >= tsp
                        rmax = jnp.maximum(rmax, v)
                        cnts.append(plsc.all_reduce_population_count(mask))
                        vs.append(v)
                        masks.append(mask)
                    offs = [off0]
                    for u in range(3):
                        offs.append(offs[u] + cnts[u][0])
                    for u in range(4):
                        off_use = jnp.minimum(offs[u], CAP - 16)
                        plsc.store_compressed(cv.at[pl.ds(off_use, 16)],
                                              vs[u], mask=masks[u])
                        plsc.store_compressed(ci.at[pl.ds(off_use, 16)],
                                              gbase + (i * 4 + u) * 16 + iota16,
                                              mask=masks[u])
                    return offs[3] + cnts[3][0], rmax

                off, rmax_v = lax.fori_loop(0, NVC // 4, filt, (off, rmax_v))

            c = jnp.minimum(off, CAP - 64)
            negs = jnp.full((16,), NEG, jnp.float32)
            for u in range(4):
                plsc.store_compressed(cv.at[pl.ds(c + u * 16, 16)], negs,
                                      mask=ones16)
            nvec = (c + 15) // 16
            ngrp = (c + 63) // 64
            rmax = jnp.max(rmax_v)

            def bstep(_, lohi):
                lo, hi = lohi
                mid = 0.5 * (lo + hi)
                mids = jnp.full((16,), mid, jnp.float32)

                def cbody(i, accs):
                    outs = []
                    for u in range(4):
                        v = cv[pl.ds((i * 4 + u) * 16, 16)]
                        outs.append(accs[u] +
                                    plsc.all_reduce_population_count(v >= mids))
                    return tuple(outs)

                z = jnp.zeros((16,), jnp.int32)
                a0, a1, a2, a3 = lax.fori_loop(0, ngrp, cbody, (z, z, z, z))
                cnt = ((a0 + a1) + (a2 + a3))[0]
                ok = cnt >= NSEL
                return jnp.where(ok, mid, lo), jnp.where(ok, hi, mid)

            lo, _ = lax.fori_loop(0, NBIS, bstep, (tau_s, rmax + jnp.float32(1.0)))

            los = jnp.full((16,), lo, jnp.float32)

            def filt2(i, off2):
                v = cv[pl.ds(i * 16, 16)]
                ix = ci[pl.ds(i * 16, 16)]
                mask = v >= los
                off_use = jnp.minimum(off2, CAP2 - 16)
                plsc.store_compressed(cv2.at[pl.ds(off_use, 16)], v, mask=mask)
                plsc.store_compressed(ci2.at[pl.ds(off_use, 16)], ix, mask=mask)
                return off2 + plsc.all_reduce_population_count(mask)[0]

            c2 = lax.fori_loop(0, nvec, filt2, jnp.int32(0))
            c2 = jnp.minimum(c2, CAP2 - 16)
            plsc.store_compressed(cv2.at[pl.ds(c2, 16)],
                                  jnp.full((16,), NEG, jnp.float32), mask=ones16)
            nvec2 = (c2 + 15) // 16

            pv = rmax + jnp.float32(1.0)
            pidx = jnp.int32(-1)
            for g in range(NSEL // 16):
                def ext(k, carry):
                    pv, pidx, selv = carry
                    pvs = jnp.full((16,), pv, jnp.float32)
                    pis = jnp.full((16,), pidx, jnp.int32)

                    def scan(i, mi):
                        mx, ix = mi
                        v = cv2[pl.ds(i * 16, 16)]
                        d = ci2[pl.ds(i * 16, 16)]
                        elig = (v < pvs) | ((v == pvs) & (d > pis))
                        better = elig & ((v > mx) | ((v == mx) & (d < ix)))
                        return jnp.where(better, v, mx), jnp.where(better, d, ix)

                    mx, ix = lax.fori_loop(
                        0, nvec2, scan,
                        (jnp.full((16,), NEG, jnp.float32),
                         jnp.full((16,), BIG, jnp.int32)))
                    m = jnp.max(mx)
                    ixm = jnp.where(mx == jnp.full((16,), m, jnp.float32), ix,
                                    jnp.full((16,), BIG, jnp.int32))
                    sel = jnp.min(ixm)
                    selv = jnp.where(iota16 == jnp.full((16,), k, jnp.int32),
                                     jnp.full((16,), sel, jnp.int32), selv)
                    return m, sel, selv

                pv, pidx, selv = lax.fori_loop(
                    0, 16, ext, (pv, pidx, jnp.zeros((16,), jnp.int32)))
                knn_v[pl.ds(g * 16, 16)] = selv

            pltpu.async_copy(cat_hbm.at[knn_v], emb_v, gsem).wait()
            pltpu.sync_copy(emb_v.at[pl.ds(0, TOPK)], out_hbm.at[b])
            return 0

        lax.fori_loop(0, ROWS_PER_W, do_row, 0)

    return sc_retrieve


_sc_retrieve = _make_sc_retrieve()


def kernel(input_video, input_text, W_video, b_video, W_text, b_text, category_embs):
    fused = pl.pallas_call(
        _fuse_body,
        out_shape=jax.ShapeDtypeStruct((B, EMB), jnp.float32),
    )(input_video, input_text, W_video, W_text,
      b_video.reshape(1, EMB), b_text.reshape(1, EMB))
    fused = fused / (jnp.linalg.norm(fused, axis=-1, keepdims=True) + 1e-6)

    pop = category_embs / (jnp.linalg.norm(category_embs, axis=-1, keepdims=True) + 1e-6)
    pop = jnp.pad(pop, ((0, POP_PAD - POP), (0, 0)))
    sims, tau2d = pl.pallas_call(
        _sims_body,
        grid=(NBLK,),
        in_specs=[
            pl.BlockSpec((B, EMB), lambda j: (0, 0)),
            pl.BlockSpec((CHUNK, EMB), lambda j: (j, 0)),
        ],
        out_specs=[
            pl.BlockSpec((B, CHUNK), lambda j: (0, j)),
            pl.BlockSpec((B, 128), lambda j: (0, 0)),
        ],
        out_shape=[
            jax.ShapeDtypeStruct((B, POP_PAD), jnp.float32),
            jax.ShapeDtypeStruct((B, 128), jnp.float32),
        ],
        scratch_shapes=[pltpu.VMEM((B, 128), jnp.float32)],
    )(fused, pop)

    tau = tau2d[:, :16]
    return _sc_retrieve(sims, tau, category_embs)
